# Initial kernel scaffold; baseline (speedup 1.0000x reference)
import jax
import jax.numpy as jnp
from jax.experimental import pallas as pl


def kernel(x, edge_index, conv_W, conv_b, ln_w, ln_b, head_w, head_b):
    raise NotImplementedError("write your pallas kernel here")


# speedup vs baseline: 5.5121x; 5.5121x over previous
import functools

import jax
import jax.numpy as jnp
from jax import lax
from jax.experimental import pallas as pl
from jax.experimental.pallas import tpu as pltpu
from jax.experimental.pallas import tpu_sc as plsc

_N = 10000
_F = 128
_NPAD = 10240
_E = 320000
_NW = 32
_CHUNK = 128
_CHUNKS = 80
_EW = _CHUNK * _CHUNKS
_EPAD = _NW * _EW
_RPT = _NPAD // 16
_IB = 16
_IBS = _CHUNKS // _IB

_mesh = plsc.VectorSubcoreMesh(core_axis_name="c", subcore_axis_name="s")


@functools.partial(
    pl.kernel,
    out_type=jax.ShapeDtypeStruct((2, _NPAD, _F), jnp.float32),
    mesh=_mesh,
    scratch_types=[
        pltpu.VMEM((_IB, _CHUNK), jnp.int32),
        pltpu.VMEM((_IB, _CHUNK), jnp.int32),
        pltpu.VMEM((2, _CHUNK, _F), jnp.float32),
        pltpu.VMEM_SHARED((_NPAD, _F), jnp.float32),
        pltpu.SemaphoreType.DMA,
    ],
)
def _sc_msgpass(g_hbm, src_hbm, dst_hbm, zeros_hbm, out_hbm,
                src_v, dst_v, rows_v, acc_sh, gsem):
    c = lax.axis_index("c")
    s = lax.axis_index("s")
    wid = s * 2 + c
    base = s * _RPT
    pltpu.sync_copy(zeros_hbm.at[pl.ds(base, _RPT)],
                    acc_sh.at[pl.ds(base, _RPT)])
    plsc.subcore_barrier()

    def outer(bi, _):
        pltpu.sync_copy(src_hbm.at[wid, pl.ds(bi * _IB, _IB)], src_v)
        pltpu.sync_copy(dst_hbm.at[wid, pl.ds(bi * _IB, _IB)], dst_v)

        def body(j, _):
            pltpu.async_copy(g_hbm.at[src_v.at[j]], rows_v.at[0], gsem).wait()
            pltpu.sync_copy(rows_v.at[0], acc_sh.at[dst_v.at[j]], add=True)
            return 0

        lax.fori_loop(0, _IB, body, 0)
        return 0

    lax.fori_loop(0, _IBS, outer, 0)
    plsc.subcore_barrier()
    pltpu.sync_copy(acc_sh.at[pl.ds(base, _RPT)],
                    out_hbm.at[c, pl.ds(base, _RPT)])


@functools.partial(
    pl.kernel,
    out_type=jax.ShapeDtypeStruct((2, _NPAD, _F), jnp.float32),
    mesh=_mesh,
    scratch_types=[
        pltpu.VMEM((_IB, _CHUNK), jnp.int32),
        pltpu.VMEM((_CHUNK, _F), jnp.float32),
        pltpu.VMEM_SHARED((_NPAD, _F), jnp.float32),
    ],
)
def _sc_deg(dst_hbm, ones_hbm, zeros_hbm, out_hbm, dst_v, ones_v, acc_sh):
    c = lax.axis_index("c")
    s = lax.axis_index("s")
    wid = s * 2 + c
    pltpu.sync_copy(ones_hbm, ones_v)
    base = s * _RPT
    pltpu.sync_copy(zeros_hbm.at[pl.ds(base, _RPT)],
                    acc_sh.at[pl.ds(base, _RPT)])
    plsc.subcore_barrier()

    def outer(bi, _):
        pltpu.sync_copy(dst_hbm.at[wid, pl.ds(bi * _IB, _IB)], dst_v)

        def body(j, _):
            pltpu.sync_copy(ones_v, acc_sh.at[dst_v.at[j]], add=True)
            return 0

        lax.fori_loop(0, _IB, body, 0)
        return 0

    lax.fori_loop(0, _IBS, outer, 0)
    plsc.subcore_barrier()
    pltpu.sync_copy(acc_sh.at[pl.ds(base, _RPT)],
                    out_hbm.at[c, pl.ds(base, _RPT)])


_BR = 512


def _tc_dinv_body(degp_ref, out_ref):
    d = degp_ref[0, :, 0:1] + degp_ref[1, :, 0:1] + 1.0
    rows = lax.broadcasted_iota(jnp.int32, (_NPAD, 1), 0)
    out_ref[...] = jnp.where(rows < _N, lax.rsqrt(d), 0.0)


_tc_dinv = pl.pallas_call(
    _tc_dinv_body,
    out_shape=jax.ShapeDtypeStruct((_NPAD, 1), jnp.float32),
)


def _tc_pre_body(x_ref, w_ref, dinv_ref, out_ref):
    out_ref[...] = jnp.dot(x_ref[...], w_ref[...],
                           preferred_element_type=jnp.float32) * dinv_ref[...]


_tc_pre = pl.pallas_call(
    _tc_pre_body,
    grid=(_NPAD // _BR,),
    in_specs=[pl.BlockSpec((_BR, _F), lambda i: (i, 0)),
              pl.BlockSpec((_F, _F), lambda i: (0, 0)),
              pl.BlockSpec((_BR, 1), lambda i: (i, 0))],
    out_specs=pl.BlockSpec((_BR, _F), lambda i: (i, 0)),
    out_shape=jax.ShapeDtypeStruct((_NPAD, _F), jnp.float32),
)


def _post_ln(p_ref, g_ref, dinv_ref, b_ref, lw_ref, lb_ref):
    t = (p_ref[0] + p_ref[1] + g_ref[...]) * dinv_ref[...] + b_ref[...]
    mu = jnp.mean(t, axis=-1, keepdims=True)
    var = jnp.mean((t - mu) ** 2, axis=-1, keepdims=True)
    return (t - mu) * lax.rsqrt(var + 1e-5) * lw_ref[...] + lb_ref[...]


def _tc_mid_body(p_ref, g_ref, dinv_ref, b_ref, lw_ref, lb_ref, w_ref, out_ref):
    h = _post_ln(p_ref, g_ref, dinv_ref, b_ref, lw_ref, lb_ref)
    a = 0.5 * h * (1.0 + lax.erf(h * 0.7071067811865476))
    out_ref[...] = jnp.dot(a, w_ref[...],
                           preferred_element_type=jnp.float32) * dinv_ref[...]


_tc_mid = pl.pallas_call(
    _tc_mid_body,
    grid=(_NPAD // _BR,),
    in_specs=[pl.BlockSpec((2, _BR, _F), lambda i: (0, i, 0)),
              pl.BlockSpec((_BR, _F), lambda i: (i, 0)),
              pl.BlockSpec((_BR, 1), lambda i: (i, 0)),
              pl.BlockSpec((1, _F), lambda i: (0, 0)),
              pl.BlockSpec((1, _F), lambda i: (0, 0)),
              pl.BlockSpec((1, _F), lambda i: (0, 0)),
              pl.BlockSpec((_F, _F), lambda i: (0, 0))],
    out_specs=pl.BlockSpec((_BR, _F), lambda i: (i, 0)),
    out_shape=jax.ShapeDtypeStruct((_NPAD, _F), jnp.float32),
)


def _tc_fin_body(p_ref, g_ref, dinv_ref, b_ref, lw_ref, lb_ref, hw_ref, hb_ref,
                 h_ref, s_ref):
    h = _post_ln(p_ref, g_ref, dinv_ref, b_ref, lw_ref, lb_ref)
    h_ref[...] = h
    s_ref[...] = jnp.dot(h, hw_ref[...],
                         preferred_element_type=jnp.float32) + hb_ref[0, 0]


_tc_fin = pl.pallas_call(
    _tc_fin_body,
    grid=(_NPAD // _BR,),
    in_specs=[pl.BlockSpec((2, _BR, _F), lambda i: (0, i, 0)),
              pl.BlockSpec((_BR, _F), lambda i: (i, 0)),
              pl.BlockSpec((_BR, 1), lambda i: (i, 0)),
              pl.BlockSpec((1, _F), lambda i: (0, 0)),
              pl.BlockSpec((1, _F), lambda i: (0, 0)),
              pl.BlockSpec((1, _F), lambda i: (0, 0)),
              pl.BlockSpec((_F, 1), lambda i: (0, 0)),
              pl.BlockSpec((1, 1), lambda i: (0, 0))],
    out_specs=[pl.BlockSpec((_BR, _F), lambda i: (i, 0)),
               pl.BlockSpec((_BR, 1), lambda i: (i, 0))],
    out_shape=[jax.ShapeDtypeStruct((_NPAD, _F), jnp.float32),
               jax.ShapeDtypeStruct((_NPAD, 1), jnp.float32)],
)


def kernel(x, edge_index, conv_W, conv_b, ln_w, ln_b, head_w, head_b):
    f32 = jnp.float32
    src = edge_index[0]
    dst = edge_index[1]
    pad = _EPAD - _E
    dummy = jnp.full((pad,), _N, jnp.int32)
    src_p = jnp.concatenate([src.astype(jnp.int32), dummy]).reshape(_NW, _CHUNKS, _CHUNK)
    dst_p = jnp.concatenate([dst.astype(jnp.int32), dummy]).reshape(_NW, _CHUNKS, _CHUNK)
    xp = jnp.pad(x, ((0, _NPAD - _N), (0, 0)))
    zeros = jnp.zeros((_NPAD, _F), f32)
    onesr = jnp.ones((_CHUNK, _F), f32)

    degp = _sc_deg(dst_p, onesr, zeros)
    dinv = _tc_dinv(degp)
    g = _tc_pre(xp, conv_W[0], dinv)
    h = s = None
    for i in range(4):
        p = _sc_msgpass(g, src_p, dst_p, zeros)
        bi = conv_b[i].reshape(1, _F)
        lwi = ln_w[i].reshape(1, _F)
        lbi = ln_b[i].reshape(1, _F)
        if i < 3:
            g = _tc_mid(p, g, dinv, bi, lwi, lbi, conv_W[i + 1])
        else:
            h, s = _tc_fin(p, g, dinv, bi, lwi, lbi, head_w, head_b.reshape(1, 1))
    return (s[:_N, 0], h[:_N])


# speedup vs baseline: 5.9376x; 1.0772x over previous
import functools

import jax
import jax.numpy as jnp
from jax import lax
from jax.experimental import pallas as pl
from jax.experimental.pallas import tpu as pltpu
from jax.experimental.pallas import tpu_sc as plsc

_N = 10000
_F = 128
_NPAD = 10240
_E = 320000
_NW = 32
_CHUNK = 128
_CHUNKS = 80
_EW = _CHUNK * _CHUNKS
_EPAD = _NW * _EW
_RPT = _NPAD // 16
_IB = 16
_IBS = _CHUNKS // _IB

_mesh = plsc.VectorSubcoreMesh(core_axis_name="c", subcore_axis_name="s")


@functools.partial(
    pl.kernel,
    out_type=jax.ShapeDtypeStruct((2, _NPAD, _F), jnp.float32),
    mesh=_mesh,
    scratch_types=[
        pltpu.VMEM((2, _IB, _CHUNK), jnp.int32),
        pltpu.VMEM((2, _IB, _CHUNK), jnp.int32),
        pltpu.VMEM((2, _CHUNK, _F), jnp.float32),
        pltpu.VMEM_SHARED((_NPAD, _F), jnp.float32),
        pltpu.SemaphoreType.DMA,
    ],
)
def _sc_msgpass(g_hbm, src_hbm, dst_hbm, zeros_hbm, out_hbm,
                src_v, dst_v, rows_v, acc_sh, gsem):
    c = lax.axis_index("c")
    s = lax.axis_index("s")
    wid = s * 2 + c
    base = s * _RPT
    pltpu.sync_copy(zeros_hbm.at[pl.ds(base, _RPT)],
                    acc_sh.at[pl.ds(base, _RPT)])
    plsc.subcore_barrier()

    def gather(nib, njj, buf):
        return pltpu.async_copy(g_hbm.at[src_v.at[nib, njj]], rows_v.at[buf],
                                gsem)

    pltpu.sync_copy(src_hbm.at[wid, pl.ds(0, _IB)], src_v.at[0])
    pltpu.sync_copy(dst_hbm.at[wid, pl.ds(0, _IB)], dst_v.at[0])
    gather(0, 0, 0)

    def outer(bi, _):
        ib = lax.rem(bi, 2)

        @pl.when(bi + 1 < _IBS)
        def _():
            nxt = lax.rem(bi + 1, 2)
            pltpu.sync_copy(src_hbm.at[wid, pl.ds((bi + 1) * _IB, _IB)],
                            src_v.at[nxt])
            pltpu.sync_copy(dst_hbm.at[wid, pl.ds((bi + 1) * _IB, _IB)],
                            dst_v.at[nxt])

        def step(t, j, buf):
            pltpu.make_async_copy(g_hbm.at[src_v.at[ib, j]], rows_v.at[buf],
                                  gsem).wait()

            @pl.when(t + 1 < _CHUNKS)
            def _():
                nib = lax.select(j + 1 < _IB, ib, lax.rem(bi + 1, 2))
                njj = lax.rem(j + 1, _IB)
                gather(nib, njj, 1 - buf)

            pltpu.sync_copy(rows_v.at[buf], acc_sh.at[dst_v.at[ib, j]],
                            add=True)

        def body(jj, _):
            j0 = 2 * jj
            t0 = bi * _IB + j0
            step(t0, j0, 0)
            step(t0 + 1, j0 + 1, 1)
            return 0

        lax.fori_loop(0, _IB // 2, body, 0)
        return 0

    lax.fori_loop(0, _IBS, outer, 0)
    plsc.subcore_barrier()
    pltpu.sync_copy(acc_sh.at[pl.ds(base, _RPT)],
                    out_hbm.at[c, pl.ds(base, _RPT)])


@functools.partial(
    pl.kernel,
    out_type=jax.ShapeDtypeStruct((2, _NPAD, _F), jnp.float32),
    mesh=_mesh,
    scratch_types=[
        pltpu.VMEM((_IB, _CHUNK), jnp.int32),
        pltpu.VMEM((_CHUNK, _F), jnp.float32),
        pltpu.VMEM_SHARED((_NPAD, _F), jnp.float32),
    ],
)
def _sc_deg(dst_hbm, ones_hbm, zeros_hbm, out_hbm, dst_v, ones_v, acc_sh):
    c = lax.axis_index("c")
    s = lax.axis_index("s")
    wid = s * 2 + c
    pltpu.sync_copy(ones_hbm, ones_v)
    base = s * _RPT
    pltpu.sync_copy(zeros_hbm.at[pl.ds(base, _RPT)],
                    acc_sh.at[pl.ds(base, _RPT)])
    plsc.subcore_barrier()

    def outer(bi, _):
        pltpu.sync_copy(dst_hbm.at[wid, pl.ds(bi * _IB, _IB)], dst_v)

        def body(j, _):
            pltpu.sync_copy(ones_v, acc_sh.at[dst_v.at[j]], add=True)
            return 0

        lax.fori_loop(0, _IB, body, 0)
        return 0

    lax.fori_loop(0, _IBS, outer, 0)
    plsc.subcore_barrier()
    pltpu.sync_copy(acc_sh.at[pl.ds(base, _RPT)],
                    out_hbm.at[c, pl.ds(base, _RPT)])


_BR = 512


def _tc_dinv_body(degp_ref, out_ref):
    d = degp_ref[0, :, 0:1] + degp_ref[1, :, 0:1] + 1.0
    rows = lax.broadcasted_iota(jnp.int32, (_NPAD, 1), 0)
    out_ref[...] = jnp.where(rows < _N, lax.rsqrt(d), 0.0)


_tc_dinv = pl.pallas_call(
    _tc_dinv_body,
    out_shape=jax.ShapeDtypeStruct((_NPAD, 1), jnp.float32),
)


def _tc_pre_body(x_ref, w_ref, dinv_ref, out_ref):
    out_ref[...] = jnp.dot(x_ref[...], w_ref[...],
                           preferred_element_type=jnp.float32) * dinv_ref[...]


_tc_pre = pl.pallas_call(
    _tc_pre_body,
    grid=(_NPAD // _BR,),
    in_specs=[pl.BlockSpec((_BR, _F), lambda i: (i, 0)),
              pl.BlockSpec((_F, _F), lambda i: (0, 0)),
              pl.BlockSpec((_BR, 1), lambda i: (i, 0))],
    out_specs=pl.BlockSpec((_BR, _F), lambda i: (i, 0)),
    out_shape=jax.ShapeDtypeStruct((_NPAD, _F), jnp.float32),
)


def _post_ln(p_ref, g_ref, dinv_ref, b_ref, lw_ref, lb_ref):
    t = (p_ref[0] + p_ref[1] + g_ref[...]) * dinv_ref[...] + b_ref[...]
    mu = jnp.mean(t, axis=-1, keepdims=True)
    var = jnp.mean((t - mu) ** 2, axis=-1, keepdims=True)
    return (t - mu) * lax.rsqrt(var + 1e-5) * lw_ref[...] + lb_ref[...]


def _tc_mid_body(p_ref, g_ref, dinv_ref, b_ref, lw_ref, lb_ref, w_ref, out_ref):
    h = _post_ln(p_ref, g_ref, dinv_ref, b_ref, lw_ref, lb_ref)
    a = 0.5 * h * (1.0 + lax.erf(h * 0.7071067811865476))
    out_ref[...] = jnp.dot(a, w_ref[...],
                           preferred_element_type=jnp.float32) * dinv_ref[...]


_tc_mid = pl.pallas_call(
    _tc_mid_body,
    grid=(_NPAD // _BR,),
    in_specs=[pl.BlockSpec((2, _BR, _F), lambda i: (0, i, 0)),
              pl.BlockSpec((_BR, _F), lambda i: (i, 0)),
              pl.BlockSpec((_BR, 1), lambda i: (i, 0)),
              pl.BlockSpec((1, _F), lambda i: (0, 0)),
              pl.BlockSpec((1, _F), lambda i: (0, 0)),
              pl.BlockSpec((1, _F), lambda i: (0, 0)),
              pl.BlockSpec((_F, _F), lambda i: (0, 0))],
    out_specs=pl.BlockSpec((_BR, _F), lambda i: (i, 0)),
    out_shape=jax.ShapeDtypeStruct((_NPAD, _F), jnp.float32),
)


def _tc_fin_body(p_ref, g_ref, dinv_ref, b_ref, lw_ref, lb_ref, hw_ref, hb_ref,
                 h_ref, s_ref):
    h = _post_ln(p_ref, g_ref, dinv_ref, b_ref, lw_ref, lb_ref)
    h_ref[...] = h
    s_ref[...] = jnp.dot(h, hw_ref[...],
                         preferred_element_type=jnp.float32) + hb_ref[0, 0]


_tc_fin = pl.pallas_call(
    _tc_fin_body,
    grid=(_NPAD // _BR,),
    in_specs=[pl.BlockSpec((2, _BR, _F), lambda i: (0, i, 0)),
              pl.BlockSpec((_BR, _F), lambda i: (i, 0)),
              pl.BlockSpec((_BR, 1), lambda i: (i, 0)),
              pl.BlockSpec((1, _F), lambda i: (0, 0)),
              pl.BlockSpec((1, _F), lambda i: (0, 0)),
              pl.BlockSpec((1, _F), lambda i: (0, 0)),
              pl.BlockSpec((_F, 1), lambda i: (0, 0)),
              pl.BlockSpec((1, 1), lambda i: (0, 0))],
    out_specs=[pl.BlockSpec((_BR, _F), lambda i: (i, 0)),
               pl.BlockSpec((_BR, 1), lambda i: (i, 0))],
    out_shape=[jax.ShapeDtypeStruct((_NPAD, _F), jnp.float32),
               jax.ShapeDtypeStruct((_NPAD, 1), jnp.float32)],
)


def kernel(x, edge_index, conv_W, conv_b, ln_w, ln_b, head_w, head_b):
    f32 = jnp.float32
    src = edge_index[0]
    dst = edge_index[1]
    pad = _EPAD - _E
    dummy = jnp.full((pad,), _N, jnp.int32)
    src_p = jnp.concatenate([src.astype(jnp.int32), dummy]).reshape(_NW, _CHUNKS, _CHUNK)
    dst_p = jnp.concatenate([dst.astype(jnp.int32), dummy]).reshape(_NW, _CHUNKS, _CHUNK)
    xp = jnp.pad(x, ((0, _NPAD - _N), (0, 0)))
    zeros = jnp.zeros((_NPAD, _F), f32)
    onesr = jnp.ones((_CHUNK, _F), f32)

    degp = _sc_deg(dst_p, onesr, zeros)
    dinv = _tc_dinv(degp)
    g = _tc_pre(xp, conv_W[0], dinv)
    h = s = None
    for i in range(4):
        p = _sc_msgpass(g, src_p, dst_p, zeros)
        bi = conv_b[i].reshape(1, _F)
        lwi = ln_w[i].reshape(1, _F)
        lbi = ln_b[i].reshape(1, _F)
        if i < 3:
            g = _tc_mid(p, g, dinv, bi, lwi, lbi, conv_W[i + 1])
        else:
            h, s = _tc_fin(p, g, dinv, bi, lwi, lbi, head_w, head_b.reshape(1, 1))
    return (s[:_N, 0], h[:_N])


# speedup vs baseline: 18.8853x; 3.1807x over previous
import functools

import jax
import jax.numpy as jnp
from jax import lax
from jax.experimental import pallas as pl
from jax.experimental.pallas import tpu as pltpu
from jax.experimental.pallas import tpu_sc as plsc

_N = 10000
_F = 128
_NPAD = 10240
_E = 320000
_NW = 32
_CHUNK = 128
_CHUNKS = 80
_EW = _CHUNK * _CHUNKS
_EPAD = _NW * _EW
_RPT = _NPAD // 16
_IB = 16
_IBS = _CHUNKS // _IB

_mesh = plsc.VectorSubcoreMesh(core_axis_name="c", subcore_axis_name="s")


@functools.partial(
    pl.kernel,
    out_type=jax.ShapeDtypeStruct((2, _NPAD, _F), jnp.float32),
    mesh=_mesh,
    scratch_types=[
        pltpu.VMEM((2, _IB, _CHUNK), jnp.int32),
        pltpu.VMEM((2, _IB, _CHUNK), jnp.int32),
        pltpu.VMEM((2, _CHUNK, _F), jnp.float32),
        pltpu.VMEM_SHARED((_NPAD, _F), jnp.float32),
        pltpu.SemaphoreType.DMA,
    ],
)
def _sc_msgpass(g_hbm, src_hbm, dst_hbm, zeros_hbm, out_hbm,
                src_v, dst_v, rows_v, acc_sh, gsem):
    c = lax.axis_index("c")
    s = lax.axis_index("s")
    wid = s * 2 + c
    base = s * _RPT
    pltpu.sync_copy(zeros_hbm.at[pl.ds(base, _RPT)],
                    acc_sh.at[pl.ds(base, _RPT)])
    plsc.subcore_barrier()

    def gather(nib, njj, buf):
        return pltpu.async_copy(g_hbm.at[src_v.at[nib, njj]], rows_v.at[buf],
                                gsem)

    pltpu.sync_copy(src_hbm.at[wid, pl.ds(0, _IB)], src_v.at[0])
    pltpu.sync_copy(dst_hbm.at[wid, pl.ds(0, _IB)], dst_v.at[0])
    gather(0, 0, 0)

    def outer(bi, _):
        ib = lax.rem(bi, 2)

        @pl.when(bi + 1 < _IBS)
        def _():
            nxt = lax.rem(bi + 1, 2)
            pltpu.sync_copy(src_hbm.at[wid, pl.ds((bi + 1) * _IB, _IB)],
                            src_v.at[nxt])
            pltpu.sync_copy(dst_hbm.at[wid, pl.ds((bi + 1) * _IB, _IB)],
                            dst_v.at[nxt])

        def step(t, j, buf):
            pltpu.make_async_copy(g_hbm.at[src_v.at[ib, j]], rows_v.at[buf],
                                  gsem).wait()

            @pl.when(t + 1 < _CHUNKS)
            def _():
                nib = lax.select(j + 1 < _IB, ib, lax.rem(bi + 1, 2))
                njj = lax.rem(j + 1, _IB)
                gather(nib, njj, 1 - buf)

            pltpu.sync_copy(rows_v.at[buf], acc_sh.at[dst_v.at[ib, j]],
                            add=True)

        def body(jj, _):
            j0 = 2 * jj
            t0 = bi * _IB + j0
            step(t0, j0, 0)
            step(t0 + 1, j0 + 1, 1)
            return 0

        lax.fori_loop(0, _IB // 2, body, 0)
        return 0

    lax.fori_loop(0, _IBS, outer, 0)
    plsc.subcore_barrier()
    pltpu.sync_copy(acc_sh.at[pl.ds(base, _RPT)],
                    out_hbm.at[c, pl.ds(base, _RPT)])


@functools.partial(
    pl.kernel,
    out_type=jax.ShapeDtypeStruct((2, _NPAD, _F), jnp.float32),
    mesh=_mesh,
    scratch_types=[
        pltpu.VMEM((_IB, _CHUNK), jnp.int32),
        pltpu.VMEM((_CHUNK, _F), jnp.float32),
        pltpu.VMEM_SHARED((_NPAD, _F), jnp.float32),
    ],
)
def _sc_deg(dst_hbm, ones_hbm, zeros_hbm, out_hbm, dst_v, ones_v, acc_sh):
    c = lax.axis_index("c")
    s = lax.axis_index("s")
    wid = s * 2 + c
    pltpu.sync_copy(ones_hbm, ones_v)
    base = s * _RPT
    pltpu.sync_copy(zeros_hbm.at[pl.ds(base, _RPT)],
                    acc_sh.at[pl.ds(base, _RPT)])
    plsc.subcore_barrier()

    def outer(bi, _):
        pltpu.sync_copy(dst_hbm.at[wid, pl.ds(bi * _IB, _IB)], dst_v)

        def body(j, _):
            pltpu.sync_copy(ones_v, acc_sh.at[dst_v.at[j]], add=True)
            return 0

        lax.fori_loop(0, _IB, body, 0)
        return 0

    lax.fori_loop(0, _IBS, outer, 0)
    plsc.subcore_barrier()
    pltpu.sync_copy(acc_sh.at[pl.ds(base, _RPT)],
                    out_hbm.at[c, pl.ds(base, _RPT)])


_BR = 512


def _tc_dinv_body(degp_ref, out_ref):
    d = degp_ref[0, :, 0:1] + degp_ref[1, :, 0:1] + 1.0
    rows = lax.broadcasted_iota(jnp.int32, (_NPAD, 1), 0)
    out_ref[...] = jnp.where(rows < _N, lax.rsqrt(d), 0.0)


_tc_dinv = pl.pallas_call(
    _tc_dinv_body,
    out_shape=jax.ShapeDtypeStruct((_NPAD, 1), jnp.float32),
)


def _tc_pre_body(x_ref, w_ref, dinv_ref, out_ref):
    out_ref[...] = jnp.dot(x_ref[...], w_ref[...],
                           preferred_element_type=jnp.float32) * dinv_ref[...]


_tc_pre = pl.pallas_call(
    _tc_pre_body,
    grid=(_NPAD // _BR,),
    in_specs=[pl.BlockSpec((_BR, _F), lambda i: (i, 0)),
              pl.BlockSpec((_F, _F), lambda i: (0, 0)),
              pl.BlockSpec((_BR, 1), lambda i: (i, 0))],
    out_specs=pl.BlockSpec((_BR, _F), lambda i: (i, 0)),
    out_shape=jax.ShapeDtypeStruct((_NPAD, _F), jnp.float32),
)


def _post_ln(p_ref, g_ref, dinv_ref, b_ref, lw_ref, lb_ref):
    t = (p_ref[0] + p_ref[1] + g_ref[...]) * dinv_ref[...] + b_ref[...]
    mu = jnp.mean(t, axis=-1, keepdims=True)
    var = jnp.mean((t - mu) ** 2, axis=-1, keepdims=True)
    return (t - mu) * lax.rsqrt(var + 1e-5) * lw_ref[...] + lb_ref[...]


def _tc_mid_body(p_ref, g_ref, dinv_ref, b_ref, lw_ref, lb_ref, w_ref, out_ref):
    h = _post_ln(p_ref, g_ref, dinv_ref, b_ref, lw_ref, lb_ref)
    a = 0.5 * h * (1.0 + lax.erf(h * 0.7071067811865476))
    out_ref[...] = jnp.dot(a, w_ref[...],
                           preferred_element_type=jnp.float32) * dinv_ref[...]


_tc_mid = pl.pallas_call(
    _tc_mid_body,
    grid=(_NPAD // _BR,),
    in_specs=[pl.BlockSpec((2, _BR, _F), lambda i: (0, i, 0)),
              pl.BlockSpec((_BR, _F), lambda i: (i, 0)),
              pl.BlockSpec((_BR, 1), lambda i: (i, 0)),
              pl.BlockSpec((1, _F), lambda i: (0, 0)),
              pl.BlockSpec((1, _F), lambda i: (0, 0)),
              pl.BlockSpec((1, _F), lambda i: (0, 0)),
              pl.BlockSpec((_F, _F), lambda i: (0, 0))],
    out_specs=pl.BlockSpec((_BR, _F), lambda i: (i, 0)),
    out_shape=jax.ShapeDtypeStruct((_NPAD, _F), jnp.float32),
)


def _tc_fin_body(p_ref, g_ref, dinv_ref, b_ref, lw_ref, lb_ref, hw_ref, hb_ref,
                 h_ref, s_ref):
    h = _post_ln(p_ref, g_ref, dinv_ref, b_ref, lw_ref, lb_ref)
    h_ref[...] = h
    s_ref[...] = jnp.dot(h, hw_ref[...],
                         preferred_element_type=jnp.float32) + hb_ref[0, 0]


_tc_fin = pl.pallas_call(
    _tc_fin_body,
    grid=(_NPAD // _BR,),
    in_specs=[pl.BlockSpec((2, _BR, _F), lambda i: (0, i, 0)),
              pl.BlockSpec((_BR, _F), lambda i: (i, 0)),
              pl.BlockSpec((_BR, 1), lambda i: (i, 0)),
              pl.BlockSpec((1, _F), lambda i: (0, 0)),
              pl.BlockSpec((1, _F), lambda i: (0, 0)),
              pl.BlockSpec((1, _F), lambda i: (0, 0)),
              pl.BlockSpec((_F, 1), lambda i: (0, 0)),
              pl.BlockSpec((1, 1), lambda i: (0, 0))],
    out_specs=[pl.BlockSpec((_BR, _F), lambda i: (i, 0)),
               pl.BlockSpec((_BR, 1), lambda i: (i, 0))],
    out_shape=[jax.ShapeDtypeStruct((_NPAD, _F), jnp.float32),
               jax.ShapeDtypeStruct((_NPAD, 1), jnp.float32)],
)


def kernel(x, edge_index, conv_W, conv_b, ln_w, ln_b, head_w, head_b):
    f32 = jnp.float32
    src = edge_index[0]
    dst = edge_index[1]
    ew = _E // _NW
    padw = _EW - ew
    fill_src = jnp.broadcast_to(
        (jnp.arange(padw, dtype=jnp.int32) * 41) % _N, (_NW, padw))
    fill_dst = jnp.full((_NW, padw), _N, jnp.int32)
    src_p = jnp.concatenate(
        [src.astype(jnp.int32).reshape(_NW, ew), fill_src],
        axis=1).reshape(_NW, _CHUNKS, _CHUNK)
    dst_p = jnp.concatenate(
        [dst.astype(jnp.int32).reshape(_NW, ew), fill_dst],
        axis=1).reshape(_NW, _CHUNKS, _CHUNK)
    xp = jnp.pad(x, ((0, _NPAD - _N), (0, 0)))
    zeros = jnp.zeros((_NPAD, _F), f32)
    onesr = jnp.ones((_CHUNK, _F), f32)

    degp = _sc_deg(dst_p, onesr, zeros)
    dinv = _tc_dinv(degp)
    g = _tc_pre(xp, conv_W[0], dinv)
    h = s = None
    for i in range(4):
        p = _sc_msgpass(g, src_p, dst_p, zeros)
        bi = conv_b[i].reshape(1, _F)
        lwi = ln_w[i].reshape(1, _F)
        lbi = ln_b[i].reshape(1, _F)
        if i < 3:
            g = _tc_mid(p, g, dinv, bi, lwi, lbi, conv_W[i + 1])
        else:
            h, s = _tc_fin(p, g, dinv, bi, lwi, lbi, head_w, head_b.reshape(1, 1))
    return (s[:_N, 0], h[:_N])


# speedup vs baseline: 18.9654x; 1.0042x over previous
import functools

import jax
import jax.numpy as jnp
from jax import lax
from jax.experimental import pallas as pl
from jax.experimental.pallas import tpu as pltpu
from jax.experimental.pallas import tpu_sc as plsc

_N = 10000
_F = 128
_NPAD = 10240
_E = 320000
_NW = 32
_CHUNK = 128
_CHUNKS = 80
_EW = _CHUNK * _CHUNKS
_EPAD = _NW * _EW
_RPT = _NPAD // 16
_IB = 16
_IBS = _CHUNKS // _IB

_mesh = plsc.VectorSubcoreMesh(core_axis_name="c", subcore_axis_name="s")


@functools.partial(
    pl.kernel,
    out_type=jax.ShapeDtypeStruct((2, _NPAD, _F), jnp.float32),
    mesh=_mesh,
    scratch_types=[
        pltpu.VMEM((2, _IB, _CHUNK), jnp.int32),
        pltpu.VMEM((2, _IB, _CHUNK), jnp.int32),
        pltpu.VMEM((2, _CHUNK, _F), jnp.float32),
        pltpu.VMEM_SHARED((_NPAD, _F), jnp.float32),
        pltpu.SemaphoreType.DMA,
    ],
)
def _sc_msgpass(g_hbm, src_hbm, dst_hbm, zeros_hbm, out_hbm,
                src_v, dst_v, rows_v, acc_sh, gsem):
    c = lax.axis_index("c")
    s = lax.axis_index("s")
    wid = s * 2 + c
    base = s * _RPT
    pltpu.sync_copy(zeros_hbm.at[pl.ds(base, _RPT)],
                    acc_sh.at[pl.ds(base, _RPT)])
    plsc.subcore_barrier()

    def gather(nib, njj, buf):
        return pltpu.async_copy(g_hbm.at[src_v.at[nib, njj]], rows_v.at[buf],
                                gsem)

    pltpu.sync_copy(src_hbm.at[wid, pl.ds(0, _IB)], src_v.at[0])
    pltpu.sync_copy(dst_hbm.at[wid, pl.ds(0, _IB)], dst_v.at[0])
    gather(0, 0, 0)

    def outer(bi, _):
        ib = lax.rem(bi, 2)

        @pl.when(bi + 1 < _IBS)
        def _():
            nxt = lax.rem(bi + 1, 2)
            pltpu.sync_copy(src_hbm.at[wid, pl.ds((bi + 1) * _IB, _IB)],
                            src_v.at[nxt])
            pltpu.sync_copy(dst_hbm.at[wid, pl.ds((bi + 1) * _IB, _IB)],
                            dst_v.at[nxt])

        def step(t, j, buf):
            pltpu.make_async_copy(g_hbm.at[src_v.at[ib, j]], rows_v.at[buf],
                                  gsem).wait()

            @pl.when(t + 1 < _CHUNKS)
            def _():
                nib = lax.select(j + 1 < _IB, ib, lax.rem(bi + 1, 2))
                njj = lax.rem(j + 1, _IB)
                gather(nib, njj, 1 - buf)

            pltpu.sync_copy(rows_v.at[buf], acc_sh.at[dst_v.at[ib, j]],
                            add=True)

        def body(jj, _):
            j0 = 2 * jj
            t0 = bi * _IB + j0
            step(t0, j0, 0)
            step(t0 + 1, j0 + 1, 1)
            return 0

        lax.fori_loop(0, _IB // 2, body, 0)
        return 0

    lax.fori_loop(0, _IBS, outer, 0)
    plsc.subcore_barrier()
    pltpu.sync_copy(acc_sh.at[pl.ds(base, _RPT)],
                    out_hbm.at[c, pl.ds(base, _RPT)])


@functools.partial(
    pl.kernel,
    out_type=jax.ShapeDtypeStruct((2, _NPAD, _F), jnp.float32),
    mesh=_mesh,
    scratch_types=[
        pltpu.VMEM((_IB, _CHUNK), jnp.int32),
        pltpu.VMEM((_CHUNK, _F), jnp.float32),
        pltpu.VMEM_SHARED((_NPAD, _F), jnp.float32),
    ],
)
def _sc_deg(dst_hbm, ones_hbm, zeros_hbm, out_hbm, dst_v, ones_v, acc_sh):
    c = lax.axis_index("c")
    s = lax.axis_index("s")
    wid = s * 2 + c
    pltpu.sync_copy(ones_hbm, ones_v)
    base = s * _RPT
    pltpu.sync_copy(zeros_hbm.at[pl.ds(base, _RPT)],
                    acc_sh.at[pl.ds(base, _RPT)])
    plsc.subcore_barrier()

    def outer(bi, _):
        pltpu.sync_copy(dst_hbm.at[wid, pl.ds(bi * _IB, _IB)], dst_v)

        def body(j, _):
            pltpu.sync_copy(ones_v, acc_sh.at[dst_v.at[j]], add=True)
            return 0

        lax.fori_loop(0, _IB, body, 0)
        return 0

    lax.fori_loop(0, _IBS, outer, 0)
    plsc.subcore_barrier()
    pltpu.sync_copy(acc_sh.at[pl.ds(base, _RPT)],
                    out_hbm.at[c, pl.ds(base, _RPT)])


_BR = 512


def _tc_pre_body(degp_ref, x_ref, w_ref, dinv_ref, g_ref):
    d = degp_ref[0, :, 0:1] + degp_ref[1, :, 0:1] + 1.0
    rows = (lax.broadcasted_iota(jnp.int32, (_BR, 1), 0)
            + pl.program_id(0) * _BR)
    dinv = jnp.where(rows < _N, lax.rsqrt(d), 0.0)
    dinv_ref[...] = dinv
    g_ref[...] = jnp.dot(x_ref[...], w_ref[...],
                         preferred_element_type=jnp.float32) * dinv


_tc_pre = pl.pallas_call(
    _tc_pre_body,
    grid=(_NPAD // _BR,),
    in_specs=[pl.BlockSpec((2, _BR, _F), lambda i: (0, i, 0)),
              pl.BlockSpec((_BR, _F), lambda i: (i, 0)),
              pl.BlockSpec((_F, _F), lambda i: (0, 0))],
    out_specs=[pl.BlockSpec((_BR, 1), lambda i: (i, 0)),
               pl.BlockSpec((_BR, _F), lambda i: (i, 0))],
    out_shape=[jax.ShapeDtypeStruct((_NPAD, 1), jnp.float32),
               jax.ShapeDtypeStruct((_NPAD, _F), jnp.float32)],
)


def _post_ln(p_ref, g_ref, dinv_ref, b_ref, lw_ref, lb_ref):
    t = (p_ref[0] + p_ref[1] + g_ref[...]) * dinv_ref[...] + b_ref[...]
    mu = jnp.mean(t, axis=-1, keepdims=True)
    var = jnp.mean((t - mu) ** 2, axis=-1, keepdims=True)
    return (t - mu) * lax.rsqrt(var + 1e-5) * lw_ref[...] + lb_ref[...]


def _tc_mid_body(p_ref, g_ref, dinv_ref, b_ref, lw_ref, lb_ref, w_ref, out_ref):
    h = _post_ln(p_ref, g_ref, dinv_ref, b_ref, lw_ref, lb_ref)
    a = 0.5 * h * (1.0 + lax.erf(h * 0.7071067811865476))
    out_ref[...] = jnp.dot(a, w_ref[...],
                           preferred_element_type=jnp.float32) * dinv_ref[...]


_tc_mid = pl.pallas_call(
    _tc_mid_body,
    grid=(_NPAD // _BR,),
    in_specs=[pl.BlockSpec((2, _BR, _F), lambda i: (0, i, 0)),
              pl.BlockSpec((_BR, _F), lambda i: (i, 0)),
              pl.BlockSpec((_BR, 1), lambda i: (i, 0)),
              pl.BlockSpec((1, _F), lambda i: (0, 0)),
              pl.BlockSpec((1, _F), lambda i: (0, 0)),
              pl.BlockSpec((1, _F), lambda i: (0, 0)),
              pl.BlockSpec((_F, _F), lambda i: (0, 0))],
    out_specs=pl.BlockSpec((_BR, _F), lambda i: (i, 0)),
    out_shape=jax.ShapeDtypeStruct((_NPAD, _F), jnp.float32),
)


def _tc_fin_body(p_ref, g_ref, dinv_ref, b_ref, lw_ref, lb_ref, hw_ref, hb_ref,
                 h_ref, s_ref):
    h = _post_ln(p_ref, g_ref, dinv_ref, b_ref, lw_ref, lb_ref)
    h_ref[...] = h
    s_ref[...] = jnp.dot(h, hw_ref[...],
                         preferred_element_type=jnp.float32) + hb_ref[0, 0]


_tc_fin = pl.pallas_call(
    _tc_fin_body,
    grid=(_NPAD // _BR,),
    in_specs=[pl.BlockSpec((2, _BR, _F), lambda i: (0, i, 0)),
              pl.BlockSpec((_BR, _F), lambda i: (i, 0)),
              pl.BlockSpec((_BR, 1), lambda i: (i, 0)),
              pl.BlockSpec((1, _F), lambda i: (0, 0)),
              pl.BlockSpec((1, _F), lambda i: (0, 0)),
              pl.BlockSpec((1, _F), lambda i: (0, 0)),
              pl.BlockSpec((_F, 1), lambda i: (0, 0)),
              pl.BlockSpec((1, 1), lambda i: (0, 0))],
    out_specs=[pl.BlockSpec((_BR, _F), lambda i: (i, 0)),
               pl.BlockSpec((_BR, 1), lambda i: (i, 0))],
    out_shape=[jax.ShapeDtypeStruct((_NPAD, _F), jnp.float32),
               jax.ShapeDtypeStruct((_NPAD, 1), jnp.float32)],
)


def kernel(x, edge_index, conv_W, conv_b, ln_w, ln_b, head_w, head_b):
    f32 = jnp.float32
    src = edge_index[0]
    dst = edge_index[1]
    ew = _E // _NW
    padw = _EW - ew
    fill_src = jnp.broadcast_to(
        (jnp.arange(padw, dtype=jnp.int32) * 41) % _N, (_NW, padw))
    fill_dst = jnp.full((_NW, padw), _N, jnp.int32)
    src_p = jnp.concatenate(
        [src.astype(jnp.int32).reshape(_NW, ew), fill_src],
        axis=1).reshape(_NW, _CHUNKS, _CHUNK)
    dst_p = jnp.concatenate(
        [dst.astype(jnp.int32).reshape(_NW, ew), fill_dst],
        axis=1).reshape(_NW, _CHUNKS, _CHUNK)
    xp = jnp.pad(x, ((0, _NPAD - _N), (0, 0)))
    zeros = jnp.zeros((_NPAD, _F), f32)
    onesr = jnp.ones((_CHUNK, _F), f32)

    degp = _sc_deg(dst_p, onesr, zeros)
    dinv, g = _tc_pre(degp, xp, conv_W[0])
    h = s = None
    for i in range(4):
        p = _sc_msgpass(g, src_p, dst_p, zeros)
        bi = conv_b[i].reshape(1, _F)
        lwi = ln_w[i].reshape(1, _F)
        lbi = ln_b[i].reshape(1, _F)
        if i < 3:
            g = _tc_mid(p, g, dinv, bi, lwi, lbi, conv_W[i + 1])
        else:
            h, s = _tc_fin(p, g, dinv, bi, lwi, lbi, head_w, head_b.reshape(1, 1))
    return (s[:_N, 0], h[:_N])


# speedup vs baseline: 18.9824x; 1.0009x over previous
import functools

import jax
import jax.numpy as jnp
from jax import lax
from jax.experimental import pallas as pl
from jax.experimental.pallas import tpu as pltpu
from jax.experimental.pallas import tpu_sc as plsc

_N = 10000
_F = 128
_NPAD = 10240
_E = 320000
_NW = 32
_CHUNK = 128
_CHUNKS = 80
_EW = _CHUNK * _CHUNKS
_EPAD = _NW * _EW
_RPT = _NPAD // 16
_IB = 16
_IBS = _CHUNKS // _IB

_mesh = plsc.VectorSubcoreMesh(core_axis_name="c", subcore_axis_name="s")


@functools.partial(
    pl.kernel,
    out_type=jax.ShapeDtypeStruct((2, _NPAD, _F), jnp.float32),
    mesh=_mesh,
    scratch_types=[
        pltpu.VMEM((2, _IB, _CHUNK), jnp.int32),
        pltpu.VMEM((2, _IB, _CHUNK), jnp.int32),
        pltpu.VMEM((2, _CHUNK, _F), jnp.float32),
        pltpu.VMEM_SHARED((_NPAD, _F), jnp.float32),
        pltpu.SemaphoreType.DMA,
        pltpu.SemaphoreType.DMA,
    ],
)
def _sc_msgpass(g_hbm, src_hbm, dst_hbm, zeros_hbm, out_hbm,
                src_v, dst_v, rows_v, acc_sh, gsem, ssem):
    c = lax.axis_index("c")
    s = lax.axis_index("s")
    wid = s * 2 + c
    base = s * _RPT
    pltpu.sync_copy(zeros_hbm.at[pl.ds(base, _RPT)],
                    acc_sh.at[pl.ds(base, _RPT)])
    plsc.subcore_barrier()

    def gather(nib, njj, buf):
        return pltpu.async_copy(g_hbm.at[src_v.at[nib, njj]], rows_v.at[buf],
                                gsem)

    pltpu.sync_copy(src_hbm.at[wid, pl.ds(0, _IB)], src_v.at[0])
    pltpu.sync_copy(dst_hbm.at[wid, pl.ds(0, _IB)], dst_v.at[0])
    gather(0, 0, 0)

    def outer(bi, _):
        ib = lax.rem(bi, 2)

        @pl.when(bi + 1 < _IBS)
        def _():
            nxt = lax.rem(bi + 1, 2)
            pltpu.sync_copy(src_hbm.at[wid, pl.ds((bi + 1) * _IB, _IB)],
                            src_v.at[nxt])
            pltpu.sync_copy(dst_hbm.at[wid, pl.ds((bi + 1) * _IB, _IB)],
                            dst_v.at[nxt])

        def step(t, j, buf):
            pltpu.make_async_copy(g_hbm.at[src_v.at[ib, j]], rows_v.at[buf],
                                  gsem).wait()
            sdesc = pltpu.async_copy(rows_v.at[buf],
                                     acc_sh.at[dst_v.at[ib, j]], ssem,
                                     add=True)

            @pl.when(t + 1 < _CHUNKS)
            def _():
                nib = lax.select(j + 1 < _IB, ib, lax.rem(bi + 1, 2))
                njj = lax.rem(j + 1, _IB)
                gather(nib, njj, 1 - buf)

            sdesc.wait()

        def body(jj, _):
            j0 = 2 * jj
            t0 = bi * _IB + j0
            step(t0, j0, 0)
            step(t0 + 1, j0 + 1, 1)
            return 0

        lax.fori_loop(0, _IB // 2, body, 0)
        return 0

    lax.fori_loop(0, _IBS, outer, 0)
    plsc.subcore_barrier()
    pltpu.sync_copy(acc_sh.at[pl.ds(base, _RPT)],
                    out_hbm.at[c, pl.ds(base, _RPT)])


@functools.partial(
    pl.kernel,
    out_type=jax.ShapeDtypeStruct((2, _NPAD, _F), jnp.float32),
    mesh=_mesh,
    scratch_types=[
        pltpu.VMEM((_IB, _CHUNK), jnp.int32),
        pltpu.VMEM((_CHUNK, _F), jnp.float32),
        pltpu.VMEM_SHARED((_NPAD, _F), jnp.float32),
    ],
)
def _sc_deg(dst_hbm, ones_hbm, zeros_hbm, out_hbm, dst_v, ones_v, acc_sh):
    c = lax.axis_index("c")
    s = lax.axis_index("s")
    wid = s * 2 + c
    pltpu.sync_copy(ones_hbm, ones_v)
    base = s * _RPT
    pltpu.sync_copy(zeros_hbm.at[pl.ds(base, _RPT)],
                    acc_sh.at[pl.ds(base, _RPT)])
    plsc.subcore_barrier()

    def outer(bi, _):
        pltpu.sync_copy(dst_hbm.at[wid, pl.ds(bi * _IB, _IB)], dst_v)

        def body(j, _):
            pltpu.sync_copy(ones_v, acc_sh.at[dst_v.at[j]], add=True)
            return 0

        lax.fori_loop(0, _IB, body, 0)
        return 0

    lax.fori_loop(0, _IBS, outer, 0)
    plsc.subcore_barrier()
    pltpu.sync_copy(acc_sh.at[pl.ds(base, _RPT)],
                    out_hbm.at[c, pl.ds(base, _RPT)])


_BR = 512


def _tc_pre_body(degp_ref, x_ref, w_ref, dinv_ref, g_ref):
    d = degp_ref[0, :, 0:1] + degp_ref[1, :, 0:1] + 1.0
    rows = (lax.broadcasted_iota(jnp.int32, (_BR, 1), 0)
            + pl.program_id(0) * _BR)
    dinv = jnp.where(rows < _N, lax.rsqrt(d), 0.0)
    dinv_ref[...] = dinv
    g_ref[...] = jnp.dot(x_ref[...], w_ref[...],
                         preferred_element_type=jnp.float32) * dinv


_tc_pre = pl.pallas_call(
    _tc_pre_body,
    grid=(_NPAD // _BR,),
    in_specs=[pl.BlockSpec((2, _BR, _F), lambda i: (0, i, 0)),
              pl.BlockSpec((_BR, _F), lambda i: (i, 0)),
              pl.BlockSpec((_F, _F), lambda i: (0, 0))],
    out_specs=[pl.BlockSpec((_BR, 1), lambda i: (i, 0)),
               pl.BlockSpec((_BR, _F), lambda i: (i, 0))],
    out_shape=[jax.ShapeDtypeStruct((_NPAD, 1), jnp.float32),
               jax.ShapeDtypeStruct((_NPAD, _F), jnp.float32)],
)


def _post_ln(p_ref, g_ref, dinv_ref, b_ref, lw_ref, lb_ref):
    t = (p_ref[0] + p_ref[1] + g_ref[...]) * dinv_ref[...] + b_ref[...]
    mu = jnp.mean(t, axis=-1, keepdims=True)
    var = jnp.mean((t - mu) ** 2, axis=-1, keepdims=True)
    return (t - mu) * lax.rsqrt(var + 1e-5) * lw_ref[...] + lb_ref[...]


def _tc_mid_body(p_ref, g_ref, dinv_ref, b_ref, lw_ref, lb_ref, w_ref, out_ref):
    h = _post_ln(p_ref, g_ref, dinv_ref, b_ref, lw_ref, lb_ref)
    a = 0.5 * h * (1.0 + lax.erf(h * 0.7071067811865476))
    out_ref[...] = jnp.dot(a, w_ref[...],
                           preferred_element_type=jnp.float32) * dinv_ref[...]


_tc_mid = pl.pallas_call(
    _tc_mid_body,
    grid=(_NPAD // _BR,),
    in_specs=[pl.BlockSpec((2, _BR, _F), lambda i: (0, i, 0)),
              pl.BlockSpec((_BR, _F), lambda i: (i, 0)),
              pl.BlockSpec((_BR, 1), lambda i: (i, 0)),
              pl.BlockSpec((1, _F), lambda i: (0, 0)),
              pl.BlockSpec((1, _F), lambda i: (0, 0)),
              pl.BlockSpec((1, _F), lambda i: (0, 0)),
              pl.BlockSpec((_F, _F), lambda i: (0, 0))],
    out_specs=pl.BlockSpec((_BR, _F), lambda i: (i, 0)),
    out_shape=jax.ShapeDtypeStruct((_NPAD, _F), jnp.float32),
)


def _tc_fin_body(p_ref, g_ref, dinv_ref, b_ref, lw_ref, lb_ref, hw_ref, hb_ref,
                 h_ref, s_ref):
    h = _post_ln(p_ref, g_ref, dinv_ref, b_ref, lw_ref, lb_ref)
    h_ref[...] = h
    s_ref[...] = jnp.dot(h, hw_ref[...],
                         preferred_element_type=jnp.float32) + hb_ref[0, 0]


_tc_fin = pl.pallas_call(
    _tc_fin_body,
    grid=(_NPAD // _BR,),
    in_specs=[pl.BlockSpec((2, _BR, _F), lambda i: (0, i, 0)),
              pl.BlockSpec((_BR, _F), lambda i: (i, 0)),
              pl.BlockSpec((_BR, 1), lambda i: (i, 0)),
              pl.BlockSpec((1, _F), lambda i: (0, 0)),
              pl.BlockSpec((1, _F), lambda i: (0, 0)),
              pl.BlockSpec((1, _F), lambda i: (0, 0)),
              pl.BlockSpec((_F, 1), lambda i: (0, 0)),
              pl.BlockSpec((1, 1), lambda i: (0, 0))],
    out_specs=[pl.BlockSpec((_BR, _F), lambda i: (i, 0)),
               pl.BlockSpec((_BR, 1), lambda i: (i, 0))],
    out_shape=[jax.ShapeDtypeStruct((_NPAD, _F), jnp.float32),
               jax.ShapeDtypeStruct((_NPAD, 1), jnp.float32)],
)


def kernel(x, edge_index, conv_W, conv_b, ln_w, ln_b, head_w, head_b):
    f32 = jnp.float32
    src = edge_index[0]
    dst = edge_index[1]
    ew = _E // _NW
    padw = _EW - ew
    fill_src = jnp.broadcast_to(
        (jnp.arange(padw, dtype=jnp.int32) * 41) % _N, (_NW, padw))
    fill_dst = jnp.full((_NW, padw), _N, jnp.int32)
    src_p = jnp.concatenate(
        [src.astype(jnp.int32).reshape(_NW, ew), fill_src],
        axis=1).reshape(_NW, _CHUNKS, _CHUNK)
    dst_p = jnp.concatenate(
        [dst.astype(jnp.int32).reshape(_NW, ew), fill_dst],
        axis=1).reshape(_NW, _CHUNKS, _CHUNK)
    xp = jnp.pad(x, ((0, _NPAD - _N), (0, 0)))
    zeros = jnp.zeros((_NPAD, _F), f32)
    onesr = jnp.ones((_CHUNK, _F), f32)

    degp = _sc_deg(dst_p, onesr, zeros)
    dinv, g = _tc_pre(degp, xp, conv_W[0])
    h = s = None
    for i in range(4):
        p = _sc_msgpass(g, src_p, dst_p, zeros)
        bi = conv_b[i].reshape(1, _F)
        lwi = ln_w[i].reshape(1, _F)
        lbi = ln_b[i].reshape(1, _F)
        if i < 3:
            g = _tc_mid(p, g, dinv, bi, lwi, lbi, conv_W[i + 1])
        else:
            h, s = _tc_fin(p, g, dinv, bi, lwi, lbi, head_w, head_b.reshape(1, 1))
    return (s[:_N, 0], h[:_N])
